# Initial kernel scaffold; baseline (speedup 1.0000x reference)
import jax
import jax.numpy as jnp
from jax.experimental import pallas as pl


def kernel(x):
    raise NotImplementedError("write your pallas kernel here")


# speedup vs baseline: 598.1112x; 598.1112x over previous
import functools

import jax
import jax.numpy as jnp
import numpy as np
from jax import lax
from jax.experimental import pallas as pl
from jax.experimental.pallas import tpu as pltpu
from jax.experimental.pallas import tpu_sc as plsc

IMG = 224
GR, GC = 232, 256
N = 32768
NIMG = 24
_L = 16


def _sc_scatter(flat, val):
    mesh = plsc.VectorSubcoreMesh(core_axis_name="c", subcore_axis_name="s")

    @functools.partial(
        pl.kernel,
        out_type=jax.ShapeDtypeStruct((NIMG * GR * GC,), jnp.float32),
        mesh=mesh,
        compiler_params=pltpu.CompilerParams(needs_layout_passes=False),
        scratch_types=[
            pltpu.VMEM((N,), jnp.int32),
            pltpu.VMEM((N,), jnp.float32),
            pltpu.VMEM((GR * GC,), jnp.float32),
        ],
    )
    def k(flat_hbm, val_hbm, out_hbm, bi, bv, grid):
        cid = lax.axis_index("c")
        sid = lax.axis_index("s")
        wid = sid * 2 + cid

        @pl.when(wid < NIMG)
        def _():
            m = wid
            pltpu.sync_copy(flat_hbm.at[pl.ds(m * N, N)], bi)
            pltpu.sync_copy(val_hbm.at[pl.ds(m * N, N)], bv)

            zv = jnp.zeros((_L,), jnp.float32)

            def z_body(i, c):
                grid[pl.ds(i * _L, _L)] = zv
                return c

            lax.fori_loop(0, GR * GC // _L, z_body, 0, unroll=8)

            def s_body(j, c):
                vidx = bi[pl.ds(j * _L, _L)]
                vv = bv[pl.ds(j * _L, _L)]
                plsc.addupdate_scatter(grid, [vidx], vv)
                return c

            lax.fori_loop(0, N // _L, s_body, 0, unroll=4)
            pltpu.sync_copy(grid, out_hbm.at[pl.ds(m * GR * GC, GR * GC)])

    return k(flat, val)


def _tc_post(G):
    def body(g_ref, o_ref):
        p = g_ref[0]
        z1r = jnp.zeros((1, GC), jnp.float32)
        z2r = jnp.zeros((2, GC), jnp.float32)
        r = (p
             + jnp.concatenate([p[1:], z1r], 0)
             + jnp.concatenate([p[2:], z2r], 0)
             + jnp.concatenate([z1r, p[:-1]], 0)
             + jnp.concatenate([z2r, p[:-2]], 0))
        z1c = jnp.zeros((GR, 1), jnp.float32)
        z2c = jnp.zeros((GR, 2), jnp.float32)
        c = (r
             + jnp.concatenate([r[:, 1:], z1c], 1)
             + jnp.concatenate([r[:, 2:], z2c], 1)
             + jnp.concatenate([z1c, r[:, :-1]], 1)
             + jnp.concatenate([z2c, r[:, :-2]], 1))
        ri = lax.broadcasted_iota(jnp.int32, (GR, GC), 0)
        ci = lax.broadcasted_iota(jnp.int32, (GR, GC), 1)
        su = jnp.concatenate([c[1:], z1r], 0)
        f1 = (su + jnp.where(ri == 0, c[0:1], 0.0)
              + jnp.where(ri == IMG - 1, c[226 - 1:226], 0.0))
        sl = jnp.concatenate([f1[:, 1:], z1c], 1)
        f2 = (sl + jnp.where(ci == 0, f1[:, 0:1], 0.0)
              + jnp.where(ci == IMG - 1, f1[:, 226 - 1:226], 0.0))
        s = f2[0:IMG, 0:IMG]
        sig = jax.nn.sigmoid(s)
        d = (sig - 0.485) / 0.229
        dmin = jnp.min(d)
        dmax = jnp.max(d)
        dn = (d - dmin) / (dmax - dmin)
        res = jnp.where(s == 0.0, -1.0, dn)
        o_ref[0] = jnp.broadcast_to(res[None], (3, IMG, IMG))

    return pl.pallas_call(
        body,
        grid=(NIMG,),
        in_specs=[pl.BlockSpec((1, GR, GC), lambda i: (i, 0, 0))],
        out_specs=pl.BlockSpec((1, 3, IMG, IMG), lambda i: (i, 0, 0, 0)),
        out_shape=jax.ShapeDtypeStruct((NIMG, 3, IMG, IMG), jnp.float32),
    )(G)


def kernel(x):
    cth = float(np.cos(-np.pi / 2))
    sth = float(np.sin(-np.pi / 2))
    rot = jnp.array([[cth, -sth, 0.0], [sth, cth, 0.0], [0.0, 0.0, 1.0]],
                    dtype=jnp.float32)
    pts1 = x @ rot
    c0, c1, c2 = pts1[..., 0], pts1[..., 1], pts1[..., 2]
    px = jnp.concatenate([c0, c2, c1], axis=0)
    py = jnp.concatenate([c1, c0, c2], axis=0)
    pv = jnp.concatenate([c2, c1, c0], axis=0)

    minx = px.min(axis=1, keepdims=True)
    maxx = px.max(axis=1, keepdims=True)
    miny = py.min(axis=1, keepdims=True)
    maxy = py.max(axis=1, keepdims=True)
    rx = maxx - minx
    ry = maxy - miny
    gs = jnp.maximum(rx, ry) / (IMG - 3)
    ix = jnp.floor((px - minx) / gs).astype(jnp.int32)
    iy = jnp.floor((py - miny) / gs).astype(jnp.int32)
    mix_ = jnp.floor(rx / gs).astype(jnp.int32)
    miy_ = jnp.floor(ry / gs).astype(jnp.int32)
    sx = 113 - ((mix_ + 2) >> 1)
    sy = 113 - ((miy_ + 2) >> 1)
    flat = (ix + sx) * GC + (iy + sy)

    g = _sc_scatter(flat.reshape(-1), pv.reshape(-1)).reshape(NIMG, GR, GC)
    return _tc_post(g)


# speedup vs baseline: 618.1428x; 1.0335x over previous
import functools

import jax
import jax.numpy as jnp
import numpy as np
from jax import lax
from jax.experimental import pallas as pl
from jax.experimental.pallas import tpu as pltpu
from jax.experimental.pallas import tpu_sc as plsc

IMG = 224
GR, GC = 232, 256
N = 32768
NIMG = 24
_L = 16


def _sc_scatter(flat, val):
    mesh = plsc.VectorSubcoreMesh(core_axis_name="c", subcore_axis_name="s")

    @functools.partial(
        pl.kernel,
        out_type=jax.ShapeDtypeStruct((NIMG * GR * GC,), jnp.float32),
        mesh=mesh,
        compiler_params=pltpu.CompilerParams(needs_layout_passes=False),
        scratch_types=[
            pltpu.VMEM((N,), jnp.int32),
            pltpu.VMEM((N,), jnp.float32),
            pltpu.VMEM((GR * GC,), jnp.float32),
            pltpu.SemaphoreType.DMA,
            pltpu.SemaphoreType.DMA,
        ],
    )
    def k(flat_hbm, val_hbm, out_hbm, bi, bv, grid, sem1, sem2):
        cid = lax.axis_index("c")
        sid = lax.axis_index("s")
        wid = sid * 2 + cid

        @pl.when(wid < NIMG)
        def _():
            m = wid
            cp1 = pltpu.async_copy(flat_hbm.at[pl.ds(m * N, N)], bi, sem1)
            cp2 = pltpu.async_copy(val_hbm.at[pl.ds(m * N, N)], bv, sem2)

            zv = jnp.zeros((_L,), jnp.float32)

            def z_body(i, c):
                grid[pl.ds(i * _L, _L)] = zv
                return c

            lax.fori_loop(0, GR * GC // _L, z_body, 0, unroll=8)
            cp1.wait()
            cp2.wait()

            def s_body(j, c):
                vidx = bi[pl.ds(j * _L, _L)]
                vv = bv[pl.ds(j * _L, _L)]
                plsc.addupdate_scatter(grid, [vidx], vv)
                return c

            lax.fori_loop(0, N // _L, s_body, 0, unroll=8)
            pltpu.sync_copy(grid, out_hbm.at[pl.ds(m * GR * GC, GR * GC)])

    return k(flat, val)


def _tc_post(G):
    def body(g_ref, o_ref):
        p = g_ref[0]
        z1r = jnp.zeros((1, GC), jnp.float32)
        z2r = jnp.zeros((2, GC), jnp.float32)
        r = (p
             + jnp.concatenate([p[1:], z1r], 0)
             + jnp.concatenate([p[2:], z2r], 0)
             + jnp.concatenate([z1r, p[:-1]], 0)
             + jnp.concatenate([z2r, p[:-2]], 0))
        z1c = jnp.zeros((GR, 1), jnp.float32)
        z2c = jnp.zeros((GR, 2), jnp.float32)
        c = (r
             + jnp.concatenate([r[:, 1:], z1c], 1)
             + jnp.concatenate([r[:, 2:], z2c], 1)
             + jnp.concatenate([z1c, r[:, :-1]], 1)
             + jnp.concatenate([z2c, r[:, :-2]], 1))
        ri = lax.broadcasted_iota(jnp.int32, (GR, GC), 0)
        ci = lax.broadcasted_iota(jnp.int32, (GR, GC), 1)
        su = jnp.concatenate([c[1:], z1r], 0)
        f1 = (su + jnp.where(ri == 0, c[0:1], 0.0)
              + jnp.where(ri == IMG - 1, c[226 - 1:226], 0.0))
        sl = jnp.concatenate([f1[:, 1:], z1c], 1)
        f2 = (sl + jnp.where(ci == 0, f1[:, 0:1], 0.0)
              + jnp.where(ci == IMG - 1, f1[:, 226 - 1:226], 0.0))
        s = f2[0:IMG, 0:IMG]
        sig = jax.nn.sigmoid(s)
        d = (sig - 0.485) / 0.229
        dmin = jnp.min(d)
        dmax = jnp.max(d)
        dn = (d - dmin) / (dmax - dmin)
        res = jnp.where(s == 0.0, -1.0, dn)
        o_ref[0] = jnp.broadcast_to(res[None], (3, IMG, IMG))

    return pl.pallas_call(
        body,
        grid=(NIMG,),
        in_specs=[pl.BlockSpec((1, GR, GC), lambda i: (i, 0, 0))],
        out_specs=pl.BlockSpec((1, 3, IMG, IMG), lambda i: (i, 0, 0, 0)),
        out_shape=jax.ShapeDtypeStruct((NIMG, 3, IMG, IMG), jnp.float32),
    )(G)


def kernel(x):
    cth = float(np.cos(-np.pi / 2))
    sth = float(np.sin(-np.pi / 2))
    rot = jnp.array([[cth, -sth, 0.0], [sth, cth, 0.0], [0.0, 0.0, 1.0]],
                    dtype=jnp.float32)
    pts1 = x @ rot
    c0, c1, c2 = pts1[..., 0], pts1[..., 1], pts1[..., 2]
    px = jnp.concatenate([c0, c2, c1], axis=0)
    py = jnp.concatenate([c1, c0, c2], axis=0)
    pv = jnp.concatenate([c2, c1, c0], axis=0)

    minx = px.min(axis=1, keepdims=True)
    maxx = px.max(axis=1, keepdims=True)
    miny = py.min(axis=1, keepdims=True)
    maxy = py.max(axis=1, keepdims=True)
    rx = maxx - minx
    ry = maxy - miny
    gs = jnp.maximum(rx, ry) / (IMG - 3)
    ix = jnp.floor((px - minx) / gs).astype(jnp.int32)
    iy = jnp.floor((py - miny) / gs).astype(jnp.int32)
    mix_ = jnp.floor(rx / gs).astype(jnp.int32)
    miy_ = jnp.floor(ry / gs).astype(jnp.int32)
    sx = 113 - ((mix_ + 2) >> 1)
    sy = 113 - ((miy_ + 2) >> 1)
    flat = (ix + sx) * GC + (iy + sy)

    g = _sc_scatter(flat.reshape(-1), pv.reshape(-1)).reshape(NIMG, GR, GC)
    return _tc_post(g)


# speedup vs baseline: 664.8268x; 1.0755x over previous
import functools

import jax
import jax.numpy as jnp
import numpy as np
from jax import lax
from jax.experimental import pallas as pl
from jax.experimental.pallas import tpu as pltpu
from jax.experimental.pallas import tpu_sc as plsc

IMG = 224
GR, GC = 232, 256
N = 32768
NIMG = 24
_L = 16


def _sc_scatter(flat, val):
    mesh = plsc.VectorSubcoreMesh(core_axis_name="c", subcore_axis_name="s")

    @functools.partial(
        pl.kernel,
        out_type=jax.ShapeDtypeStruct((NIMG * GR * GC,), jnp.float32),
        mesh=mesh,
        compiler_params=pltpu.CompilerParams(needs_layout_passes=False),
        scratch_types=[
            pltpu.VMEM((N,), jnp.int32),
            pltpu.VMEM((N,), jnp.float32),
            pltpu.VMEM((GR * GC,), jnp.float32),
            pltpu.SemaphoreType.DMA,
            pltpu.SemaphoreType.DMA,
        ],
    )
    def k(flat_hbm, val_hbm, out_hbm, bi, bv, grid, sem1, sem2):
        cid = lax.axis_index("c")
        sid = lax.axis_index("s")
        wid = sid * 2 + cid

        @pl.when(wid < NIMG)
        def _():
            m = wid
            cp1 = pltpu.async_copy(flat_hbm.at[pl.ds(m * N, N)], bi, sem1)
            cp2 = pltpu.async_copy(val_hbm.at[pl.ds(m * N, N)], bv, sem2)

            zv = jnp.zeros((_L,), jnp.float32)

            def z_body(i, c):
                grid[pl.ds(i * _L, _L)] = zv
                return c

            lax.fori_loop(0, GR * GC // _L, z_body, 0, unroll=8)
            cp1.wait()
            cp2.wait()

            def s_body(j, c):
                vidx = bi[pl.ds(j * _L, _L)]
                vv = bv[pl.ds(j * _L, _L)]
                plsc.addupdate_scatter(grid, [vidx], vv)
                return c

            lax.fori_loop(0, N // _L, s_body, 0, unroll=8)
            pltpu.sync_copy(grid, out_hbm.at[pl.ds(m * GR * GC, GR * GC)])

    return k(flat, val)


def _tc_post(G):
    def body(g_ref, o_ref):
      for bimg in range(2):
        p = g_ref[bimg]
        z1r = jnp.zeros((1, GC), jnp.float32)
        z2r = jnp.zeros((2, GC), jnp.float32)
        r = (p
             + jnp.concatenate([p[1:], z1r], 0)
             + jnp.concatenate([p[2:], z2r], 0)
             + jnp.concatenate([z1r, p[:-1]], 0)
             + jnp.concatenate([z2r, p[:-2]], 0))
        z1c = jnp.zeros((GR, 1), jnp.float32)
        z2c = jnp.zeros((GR, 2), jnp.float32)
        c = (r
             + jnp.concatenate([r[:, 1:], z1c], 1)
             + jnp.concatenate([r[:, 2:], z2c], 1)
             + jnp.concatenate([z1c, r[:, :-1]], 1)
             + jnp.concatenate([z2c, r[:, :-2]], 1))
        ri = lax.broadcasted_iota(jnp.int32, (GR, GC), 0)
        ci = lax.broadcasted_iota(jnp.int32, (GR, GC), 1)
        su = jnp.concatenate([c[1:], z1r], 0)
        f1 = (su + jnp.where(ri == 0, c[0:1], 0.0)
              + jnp.where(ri == IMG - 1, c[226 - 1:226], 0.0))
        sl = jnp.concatenate([f1[:, 1:], z1c], 1)
        f2 = (sl + jnp.where(ci == 0, f1[:, 0:1], 0.0)
              + jnp.where(ci == IMG - 1, f1[:, 226 - 1:226], 0.0))
        s = f2[0:IMG, 0:IMG]
        sig = jax.nn.sigmoid(s)
        d = (sig - 0.485) / 0.229
        dmin = jnp.min(d)
        dmax = jnp.max(d)
        dn = (d - dmin) / (dmax - dmin)
        res = jnp.where(s == 0.0, -1.0, dn)
        o_ref[bimg] = jnp.broadcast_to(res[None], (3, IMG, IMG))

    return pl.pallas_call(
        body,
        grid=(NIMG // 2,),
        in_specs=[pl.BlockSpec((2, GR, GC), lambda i: (i, 0, 0))],
        out_specs=pl.BlockSpec((2, 3, IMG, IMG), lambda i: (i, 0, 0, 0)),
        out_shape=jax.ShapeDtypeStruct((NIMG, 3, IMG, IMG), jnp.float32),
    )(G)


def kernel(x):
    cth = float(np.cos(-np.pi / 2))
    sth = float(np.sin(-np.pi / 2))
    rot = jnp.array([[cth, -sth, 0.0], [sth, cth, 0.0], [0.0, 0.0, 1.0]],
                    dtype=jnp.float32)
    pts1 = x @ rot
    c0, c1, c2 = pts1[..., 0], pts1[..., 1], pts1[..., 2]
    px = jnp.concatenate([c0, c2, c1], axis=0)
    py = jnp.concatenate([c1, c0, c2], axis=0)
    pv = jnp.concatenate([c2, c1, c0], axis=0)

    minx = px.min(axis=1, keepdims=True)
    maxx = px.max(axis=1, keepdims=True)
    miny = py.min(axis=1, keepdims=True)
    maxy = py.max(axis=1, keepdims=True)
    rx = maxx - minx
    ry = maxy - miny
    gs = jnp.maximum(rx, ry) / (IMG - 3)
    ix = jnp.floor((px - minx) / gs).astype(jnp.int32)
    iy = jnp.floor((py - miny) / gs).astype(jnp.int32)
    mix_ = jnp.floor(rx / gs).astype(jnp.int32)
    miy_ = jnp.floor(ry / gs).astype(jnp.int32)
    sx = 113 - ((mix_ + 2) >> 1)
    sy = 113 - ((miy_ + 2) >> 1)
    flat = (ix + sx) * GC + (iy + sy)

    g = _sc_scatter(flat.reshape(-1), pv.reshape(-1)).reshape(NIMG, GR, GC)
    return _tc_post(g)


# speedup vs baseline: 670.1298x; 1.0080x over previous
import functools

import jax
import jax.numpy as jnp
import numpy as np
from jax import lax
from jax.experimental import pallas as pl
from jax.experimental.pallas import tpu as pltpu
from jax.experimental.pallas import tpu_sc as plsc

IMG = 224
GR, GC = 232, 256
N = 32768
NIMG = 24
_L = 16


def _sc_scatter(flat, val):
    mesh = plsc.VectorSubcoreMesh(core_axis_name="c", subcore_axis_name="s")

    @functools.partial(
        pl.kernel,
        out_type=jax.ShapeDtypeStruct((NIMG * GR * GC,), jnp.float32),
        mesh=mesh,
        compiler_params=pltpu.CompilerParams(needs_layout_passes=False),
        scratch_types=[
            pltpu.VMEM((N,), jnp.int32),
            pltpu.VMEM((N,), jnp.float32),
            pltpu.VMEM((GR * GC,), jnp.float32),
            pltpu.SemaphoreType.DMA,
            pltpu.SemaphoreType.DMA,
        ],
    )
    def k(flat_hbm, val_hbm, out_hbm, bi, bv, grid, sem1, sem2):
        cid = lax.axis_index("c")
        sid = lax.axis_index("s")
        wid = sid * 2 + cid

        @pl.when(wid < NIMG)
        def _():
            m = wid
            cp1 = pltpu.async_copy(flat_hbm.at[pl.ds(m * N, N)], bi, sem1)
            cp2 = pltpu.async_copy(val_hbm.at[pl.ds(m * N, N)], bv, sem2)

            zv = jnp.zeros((_L,), jnp.float32)

            def z_body(i, c):
                grid[pl.ds(i * _L, _L)] = zv
                return c

            lax.fori_loop(0, GR * GC // _L, z_body, 0, unroll=8)
            cp1.wait()
            cp2.wait()

            def s_body(j, c):
                vidx = bi[pl.ds(j * _L, _L)]
                vv = bv[pl.ds(j * _L, _L)]
                plsc.addupdate_scatter(grid, [vidx], vv)
                return c

            lax.fori_loop(0, N // _L, s_body, 0, unroll=8)
            pltpu.sync_copy(grid, out_hbm.at[pl.ds(m * GR * GC, GR * GC)])

    return k(flat, val)


def _tc_post(G):
    def body(g_ref, o_ref):
      for bimg in range(4):
        p = g_ref[bimg]
        z1r = jnp.zeros((1, GC), jnp.float32)
        z2r = jnp.zeros((2, GC), jnp.float32)
        r = (p
             + jnp.concatenate([p[1:], z1r], 0)
             + jnp.concatenate([p[2:], z2r], 0)
             + jnp.concatenate([z1r, p[:-1]], 0)
             + jnp.concatenate([z2r, p[:-2]], 0))
        z1c = jnp.zeros((GR, 1), jnp.float32)
        z2c = jnp.zeros((GR, 2), jnp.float32)
        c = (r
             + jnp.concatenate([r[:, 1:], z1c], 1)
             + jnp.concatenate([r[:, 2:], z2c], 1)
             + jnp.concatenate([z1c, r[:, :-1]], 1)
             + jnp.concatenate([z2c, r[:, :-2]], 1))
        ri = lax.broadcasted_iota(jnp.int32, (GR, GC), 0)
        ci = lax.broadcasted_iota(jnp.int32, (GR, GC), 1)
        su = jnp.concatenate([c[1:], z1r], 0)
        f1 = (su + jnp.where(ri == 0, c[0:1], 0.0)
              + jnp.where(ri == IMG - 1, c[226 - 1:226], 0.0))
        sl = jnp.concatenate([f1[:, 1:], z1c], 1)
        f2 = (sl + jnp.where(ci == 0, f1[:, 0:1], 0.0)
              + jnp.where(ci == IMG - 1, f1[:, 226 - 1:226], 0.0))
        s = f2[0:IMG, 0:IMG]
        sig = jax.nn.sigmoid(s)
        d = (sig - 0.485) / 0.229
        dmin = jnp.min(d)
        dmax = jnp.max(d)
        dn = (d - dmin) / (dmax - dmin)
        res = jnp.where(s == 0.0, -1.0, dn)
        o_ref[bimg] = jnp.broadcast_to(res[None], (3, IMG, IMG))

    return pl.pallas_call(
        body,
        grid=(NIMG // 4,),
        in_specs=[pl.BlockSpec((4, GR, GC), lambda i: (i, 0, 0))],
        out_specs=pl.BlockSpec((4, 3, IMG, IMG), lambda i: (i, 0, 0, 0)),
        out_shape=jax.ShapeDtypeStruct((NIMG, 3, IMG, IMG), jnp.float32),
    )(G)


def kernel(x):
    cth = float(np.cos(-np.pi / 2))
    sth = float(np.sin(-np.pi / 2))
    rot = jnp.array([[cth, -sth, 0.0], [sth, cth, 0.0], [0.0, 0.0, 1.0]],
                    dtype=jnp.float32)
    pts1 = x @ rot
    c0, c1, c2 = pts1[..., 0], pts1[..., 1], pts1[..., 2]
    px = jnp.concatenate([c0, c2, c1], axis=0)
    py = jnp.concatenate([c1, c0, c2], axis=0)
    pv = jnp.concatenate([c2, c1, c0], axis=0)

    minx = px.min(axis=1, keepdims=True)
    maxx = px.max(axis=1, keepdims=True)
    miny = py.min(axis=1, keepdims=True)
    maxy = py.max(axis=1, keepdims=True)
    rx = maxx - minx
    ry = maxy - miny
    gs = jnp.maximum(rx, ry) / (IMG - 3)
    ix = jnp.floor((px - minx) / gs).astype(jnp.int32)
    iy = jnp.floor((py - miny) / gs).astype(jnp.int32)
    mix_ = jnp.floor(rx / gs).astype(jnp.int32)
    miy_ = jnp.floor(ry / gs).astype(jnp.int32)
    sx = 113 - ((mix_ + 2) >> 1)
    sy = 113 - ((miy_ + 2) >> 1)
    flat = (ix + sx) * GC + (iy + sy)

    g = _sc_scatter(flat.reshape(-1), pv.reshape(-1)).reshape(NIMG, GR, GC)
    return _tc_post(g)
